# Initial kernel scaffold; baseline (speedup 1.0000x reference)
import jax
import jax.numpy as jnp
from jax.experimental import pallas as pl


def kernel(x, d_Wi1, d_Wh1, d_b1, d_Wi2, d_Wh2, d_b2, d_Wo, d_bo, g_W, g_b, e_Wi1, e_Wh1, e_b1, e_Wi2, e_Wh2, e_b2, e_Wo, e_bo):
    raise NotImplementedError("write your pallas kernel here")


# speedup vs baseline: 3.6004x; 3.6004x over previous
import jax
import jax.numpy as jnp
from jax.experimental import pallas as pl

B = 32
T = 4
H = 512
E = 8


def _cell(gx, gh, b, c):
    g = gx + gh + b
    i = jax.nn.sigmoid(g[:, :H])
    f = jax.nn.sigmoid(g[:, H:2 * H])
    gg = jnp.tanh(g[:, 2 * H:3 * H])
    o = jax.nn.sigmoid(g[:, 3 * H:])
    cn = f * c + i * gg
    hn = o * jnp.tanh(cn)
    return hn, cn


def _dot(a, b):
    return jax.lax.dot_general(a, b, (((1,), (0,)), ((), ())),
                               preferred_element_type=jnp.float32)


def _moe_kernel(x0_ref,
                d_Wi1_ref, d_Wh1_ref, d_b1_ref, d_Wi2_ref, d_Wh2_ref,
                d_b2_ref,
                g_W_ref, g_b_ref,
                e_Wi1_ref, e_Wh1_ref, e_b1_ref, e_Wi2_ref, e_Wh2_ref,
                e_b2_ref, e_Wo_ref, e_bo_ref,
                out_ref):
    x0 = x0_ref[...]
    d_b1 = d_b1_ref[...]
    d_b2 = d_b2_ref[...]
    e_b1 = e_b1_ref[...]
    e_b2 = e_b2_ref[...]
    e_bo = e_bo_ref[...]
    g_b = g_b_ref[...]

    z = jnp.zeros((B, H), dtype=jnp.float32)
    d_h1, d_c1, d_h2, d_c2 = z, z, z, z
    e_h1, e_c1, e_h2, e_c2 = z, z, z, z
    o = x0

    lane = jax.lax.broadcasted_iota(jnp.int32, (B, E), 1)

    for t in range(T):
        xt = x0 if t == 0 else o

        d_h1, d_c1 = _cell(_dot(xt, d_Wi1_ref[...]),
                           _dot(d_h1, d_Wh1_ref[...]), d_b1, d_c1)
        d_h2, d_c2 = _cell(_dot(d_h1, d_Wi2_ref[...]),
                           _dot(d_h2, d_Wh2_ref[...]), d_b2, d_c2)

        logits = _dot(d_c2, g_W_ref[...]) + g_b
        m = jnp.max(logits, axis=1, keepdims=True)
        ex = jnp.exp(logits - m)
        p = ex / jnp.sum(ex, axis=1, keepdims=True)
        m1 = jnp.max(p, axis=1, keepdims=True)
        i1 = jnp.min(jnp.where(p == m1, lane, E), axis=1, keepdims=True)
        p2 = jnp.where(lane == i1, -1.0, p)
        m2 = jnp.max(p2, axis=1, keepdims=True)
        i2 = jnp.min(jnp.where(p2 == m2, lane, E), axis=1, keepdims=True)
        w_lo = jnp.where(i1 < i2, m1, m2)
        w_hi = jnp.where(i1 < i2, m2, m1)

        xw = _dot(xt, e_Wi1_ref[...])
        h1a, c1a = _cell(xw, _dot(e_h1, e_Wh1_ref[...]), e_b1, e_c1)
        h2a, c2a = _cell(_dot(h1a, e_Wi2_ref[...]),
                         _dot(e_h2, e_Wh2_ref[...]), e_b2, e_c2)
        out_a = _dot(h2a, e_Wo_ref[...]) + e_bo

        h1b, c1b = _cell(xw, _dot(h1a, e_Wh1_ref[...]), e_b1, c1a)
        h2b, c2b = _cell(_dot(h1b, e_Wi2_ref[...]),
                         _dot(h2a, e_Wh2_ref[...]), e_b2, c2a)
        out_b = _dot(h2b, e_Wo_ref[...]) + e_bo

        e_h1, e_c1, e_h2, e_c2 = h1b, c1b, h2b, c2b

        o = w_lo * out_a + w_hi * out_b
        out_ref[:, t * H:(t + 1) * H] = o


def kernel(x, d_Wi1, d_Wh1, d_b1, d_Wi2, d_Wh2, d_b2, d_Wo, d_bo,
           g_W, g_b,
           e_Wi1, e_Wh1, e_b1, e_Wi2, e_Wh2, e_b2, e_Wo, e_bo):
    del d_Wo, d_bo
    x0 = x[:, 0, :]
    out = pl.pallas_call(
        _moe_kernel,
        out_shape=jax.ShapeDtypeStruct((B, T * H), jnp.float32),
    )(x0,
      d_Wi1, d_Wh1, d_b1.reshape(1, -1), d_Wi2, d_Wh2, d_b2.reshape(1, -1),
      g_W, g_b.reshape(1, -1),
      e_Wi1, e_Wh1, e_b1.reshape(1, -1), e_Wi2, e_Wh2, e_b2.reshape(1, -1),
      e_Wo, e_bo.reshape(1, -1))
    return out.reshape(B, T, H)
